# Initial kernel scaffold; baseline (speedup 1.0000x reference)
import jax
import jax.numpy as jnp
from jax.experimental import pallas as pl


def kernel(x, edge_index, W1, b1, W_mu, b_mu, W_ls, b_ls):
    raise NotImplementedError("write your pallas kernel here")


# speedup vs baseline: 11.2667x; 11.2667x over previous
import functools

import jax
import jax.numpy as jnp
from jax import lax
from jax.experimental import pallas as pl
from jax.experimental.pallas import tpu as pltpu
from jax.experimental.pallas import tpu_sc as plsc

N = 10000
E = 320000
IN_C = 128
OUT_C = 64
HID = 128

NC = 2
NS = 16
NW = NC * NS

B = 128
NBLK = 79
EP = NW * NBLK * B
NP = NBLK * 128
ROWS_PER_TILE = NP // NS
DUMMY = N + 8

_mesh = plsc.VectorSubcoreMesh(core_axis_name="c", subcore_axis_name="s")


@functools.partial(
    pl.kernel,
    out_type=jax.ShapeDtypeStruct((NC * NP,), jnp.float32),
    mesh=_mesh,
    scratch_types=[
        pltpu.VMEM((B,), jnp.int32),
        pltpu.VMEM((B,), jnp.float32),
        pltpu.VMEM((ROWS_PER_TILE + 8,), jnp.float32),
        pltpu.VMEM_SHARED((NP,), jnp.float32),
        pltpu.SemaphoreType.DMA,
    ],
)
def _deg_kernel(dst_hbm, out_hbm, didx, ones_v, zeros_v, acc, sem):
    cc = lax.axis_index("c")
    ss = lax.axis_index("s")
    wid = cc * NS + ss

    for k in range((B + 15) // 16):
        ones_v[pl.ds(16 * k, 16)] = jnp.ones((16,), jnp.float32)
    for k in range((ROWS_PER_TILE + 8) // 16):
        zeros_v[pl.ds(16 * k, 16)] = jnp.zeros((16,), jnp.float32)
    pltpu.sync_copy(zeros_v.at[pl.ds(0, ROWS_PER_TILE)],
                    acc.at[pl.ds(ss * ROWS_PER_TILE, ROWS_PER_TILE)])
    plsc.subcore_barrier()

    def body(i, carry):
        base = wid * (NBLK * B) + i * B
        pltpu.sync_copy(dst_hbm.at[pl.ds(base, B)], didx)
        pltpu.sync_copy(ones_v, acc.at[didx], add=True)
        return carry

    lax.fori_loop(0, NBLK, body, 0)
    plsc.subcore_barrier()
    pltpu.sync_copy(acc.at[pl.ds(ss * ROWS_PER_TILE, ROWS_PER_TILE)],
                    zeros_v.at[pl.ds(0, ROWS_PER_TILE)])
    pltpu.sync_copy(zeros_v.at[pl.ds(0, ROWS_PER_TILE)],
                    out_hbm.at[pl.ds(cc * NP + ss * ROWS_PER_TILE, ROWS_PER_TILE)])


@functools.partial(
    pl.kernel,
    out_type=jax.ShapeDtypeStruct((NC, NP, HID), jnp.float32),
    mesh=_mesh,
    scratch_types=[
        pltpu.VMEM((B,), jnp.int32),
        pltpu.VMEM((B,), jnp.int32),
        pltpu.VMEM((B, HID), jnp.float32),
        pltpu.VMEM_SHARED((NP, HID), jnp.float32),
        pltpu.SemaphoreType.DMA,
    ],
)
def _edge_kernel(src_hbm, dst_hbm, hs_hbm, out_hbm, sidx, didx, rows, acc, sem):
    cc = lax.axis_index("c")
    ss = lax.axis_index("s")
    wid = cc * NS + ss
    r0 = ss * ROWS_PER_TILE

    for off, sz in ((0, 128), (128, 128), (256, 128), (384, 128), (512, 120)):
        pltpu.sync_copy(hs_hbm.at[pl.ds(r0 + off, sz)], rows.at[pl.ds(0, sz)])
        pltpu.sync_copy(rows.at[pl.ds(0, sz)], acc.at[pl.ds(r0 + off, sz)])
    plsc.subcore_barrier()

    def body(i, carry):
        base = wid * (NBLK * B) + i * B
        pltpu.sync_copy(src_hbm.at[pl.ds(base, B)], sidx)
        pltpu.sync_copy(dst_hbm.at[pl.ds(base, B)], didx)
        pltpu.async_copy(hs_hbm.at[sidx], rows, sem).wait()
        pltpu.sync_copy(rows, acc.at[didx], add=True)
        return carry

    lax.fori_loop(0, NBLK, body, 0)
    plsc.subcore_barrier()
    for off, sz in ((0, 128), (128, 128), (256, 128), (384, 128), (512, 120)):
        pltpu.sync_copy(acc.at[pl.ds(r0 + off, sz)], rows.at[pl.ds(0, sz)])
        pltpu.sync_copy(rows.at[pl.ds(0, sz)],
                        out_hbm.at[cc, pl.ds(r0 + off, sz)])


def _rowscale(dinv_row):
    m = jnp.broadcast_to(dinv_row, (128, 128))
    ii = lax.broadcasted_iota(jnp.int32, (128, 128), 0)
    jj = lax.broadcasted_iota(jnp.int32, (128, 128), 1)
    d = jnp.where(ii == jj, m, 0.0)
    return jnp.dot(d, jnp.ones((128, 128), jnp.float32),
                   preferred_element_type=jnp.float32)


def _ka_body(deg_ref, x_ref, w_ref, dinvb_ref, hs_ref):
    deg = deg_ref[:, 0, :] + deg_ref[:, 1, :] + 1.0
    dinvb = _rowscale(lax.rsqrt(deg))
    dinvb_ref[...] = dinvb
    xw = jnp.dot(x_ref[...], w_ref[...], preferred_element_type=jnp.float32)
    hs_ref[...] = dinvb * xw


def _kb_body(p_ref, hs1_ref, dinvb_ref, w_ref, b_ref, hs2_ref):
    agg = p_ref[0] + p_ref[1] - hs1_ref[...]
    h = jax.nn.relu(dinvb_ref[...] * agg + b_ref[...])
    hs2_ref[...] = dinvb_ref[...] * jnp.dot(
        h, w_ref[...], preferred_element_type=jnp.float32)


def _kc_body(q_ref, hs2_ref, dinvb_ref, b_ref, out_ref):
    agg = q_ref[0] + q_ref[1] - hs2_ref[...]
    out_ref[...] = dinvb_ref[...] * agg + b_ref[...]


_ka = pl.pallas_call(
    _ka_body,
    grid=(NBLK,),
    in_specs=[
        pl.BlockSpec((1, 2, 128), lambda i: (i, 0, 0)),
        pl.BlockSpec((128, IN_C), lambda i: (i, 0)),
        pl.BlockSpec((IN_C, HID), lambda i: (0, 0)),
    ],
    out_specs=[
        pl.BlockSpec((128, 128), lambda i: (i, 0)),
        pl.BlockSpec((128, HID), lambda i: (i, 0)),
    ],
    out_shape=[
        jax.ShapeDtypeStruct((NP, 128), jnp.float32),
        jax.ShapeDtypeStruct((NP, HID), jnp.float32),
    ],
)

_kb = pl.pallas_call(
    _kb_body,
    grid=(NBLK,),
    in_specs=[
        pl.BlockSpec((2, 128, HID), lambda i: (0, i, 0)),
        pl.BlockSpec((128, HID), lambda i: (i, 0)),
        pl.BlockSpec((128, 128), lambda i: (i, 0)),
        pl.BlockSpec((HID, 128), lambda i: (0, 0)),
        pl.BlockSpec((1, HID), lambda i: (0, 0)),
    ],
    out_specs=pl.BlockSpec((128, 128), lambda i: (i, 0)),
    out_shape=jax.ShapeDtypeStruct((NP, 128), jnp.float32),
)

_kc = pl.pallas_call(
    _kc_body,
    grid=(NBLK,),
    in_specs=[
        pl.BlockSpec((2, 128, 128), lambda i: (0, i, 0)),
        pl.BlockSpec((128, 128), lambda i: (i, 0)),
        pl.BlockSpec((128, 128), lambda i: (i, 0)),
        pl.BlockSpec((1, 128), lambda i: (0, 0)),
    ],
    out_specs=pl.BlockSpec((128, 128), lambda i: (i, 0)),
    out_shape=jax.ShapeDtypeStruct((NP, 128), jnp.float32),
)


def kernel(x, edge_index, W1, b1, W_mu, b_mu, W_ls, b_ls):
    src = edge_index[0]
    dst = edge_index[1]
    pad = EP - E
    srcp = jnp.concatenate([src, jnp.zeros((pad,), src.dtype)])
    dstp = jnp.concatenate([dst, jnp.full((pad,), DUMMY, dst.dtype)])

    degp = _deg_kernel(dstp)
    degt = jnp.transpose(degp.reshape(NC, NBLK, 128), (1, 0, 2))

    xpad = jnp.pad(x, ((0, NP - N), (0, 0)))
    dinvb, hs1 = _ka(degt, xpad, W1)

    p = _edge_kernel(srcp, dstp, hs1)

    wcat = jnp.concatenate([W_mu, W_ls], axis=1)
    hs2 = _kb(p, hs1, dinvb, wcat, b1.reshape(1, HID))

    q = _edge_kernel(srcp, dstp, hs2)

    bcat = jnp.concatenate([b_mu, b_ls]).reshape(1, 128)
    outc = _kc(q, hs2, dinvb, bcat)

    return outc[:N, :OUT_C], outc[:N, OUT_C:]
